# baseline (speedup 1.0000x reference)
import jax
import jax.numpy as jnp
from jax.experimental import pallas as pl
from jax.experimental.pallas import tpu as pltpu

_SEQ_BLK = 1024


def _make_kernel(batch, seq_len, dim, blk):
    n = seq_len // blk

    def body(w_hbm, out_hbm, buf, rsem, wsem):
        reads = [
            pltpu.make_async_copy(
                w_hbm.at[pl.ds(i * blk, blk), :], buf.at[i % 2], rsem.at[i]
            )
            for i in range(n)
        ]
        writes = [
            [
                pltpu.make_async_copy(
                    buf.at[i % 2],
                    out_hbm.at[b, pl.ds(i * blk, blk), :],
                    wsem.at[i, b],
                )
                for b in range(batch)
            ]
            for i in range(n)
        ]
        reads[0].start()
        for i in range(n):
            reads[i].wait()
            for c in writes[i]:
                c.start()
            if i + 1 < n:
                if i >= 1:
                    for c in writes[i - 1]:
                        c.wait()
                reads[i + 1].start()
        for c in writes[n - 2]:
            c.wait()
        for c in writes[n - 1]:
            c.wait()

    return body


def kernel(input_ids, emb_weight):
    batch, seq_len = input_ids.shape
    dim = emb_weight.shape[1]
    blk = _SEQ_BLK
    n = seq_len // blk
    return pl.pallas_call(
        _make_kernel(batch, seq_len, dim, blk),
        in_specs=[pl.BlockSpec(memory_space=pltpu.MemorySpace.HBM)],
        out_specs=pl.BlockSpec(memory_space=pltpu.MemorySpace.HBM),
        out_shape=jax.ShapeDtypeStruct((batch, seq_len, dim), emb_weight.dtype),
        scratch_shapes=[
            pltpu.MemorySpace.VMEM((2, blk, dim), jnp.float32),
            pltpu.SemaphoreType.DMA((n,)),
            pltpu.SemaphoreType.DMA((n, batch)),
        ],
    )(emb_weight)


# speedup vs baseline: 1.0171x; 1.0171x over previous
import jax
import jax.numpy as jnp
from jax.experimental import pallas as pl
from jax.experimental.pallas import tpu as pltpu

_SEQ_BLK = 2048


def _make_kernel(batch, seq_len, dim, blk):
    n = seq_len // blk

    def body(w_hbm, out_hbm, buf, rsem, wsem):
        reads = [
            pltpu.make_async_copy(
                w_hbm.at[pl.ds(i * blk, blk), :], buf.at[i % 2], rsem.at[i]
            )
            for i in range(n)
        ]
        writes = [
            [
                pltpu.make_async_copy(
                    buf.at[i % 2],
                    out_hbm.at[b, pl.ds(i * blk, blk), :],
                    wsem.at[i, b],
                )
                for b in range(batch)
            ]
            for i in range(n)
        ]
        reads[0].start()
        for i in range(n):
            reads[i].wait()
            for c in writes[i]:
                c.start()
            if i + 1 < n:
                if i >= 1:
                    for c in writes[i - 1]:
                        c.wait()
                reads[i + 1].start()
        for c in writes[n - 2]:
            c.wait()
        for c in writes[n - 1]:
            c.wait()

    return body


def kernel(input_ids, emb_weight):
    batch, seq_len = input_ids.shape
    dim = emb_weight.shape[1]
    blk = _SEQ_BLK
    n = seq_len // blk
    return pl.pallas_call(
        _make_kernel(batch, seq_len, dim, blk),
        in_specs=[pl.BlockSpec(memory_space=pltpu.MemorySpace.HBM)],
        out_specs=pl.BlockSpec(memory_space=pltpu.MemorySpace.HBM),
        out_shape=jax.ShapeDtypeStruct((batch, seq_len, dim), emb_weight.dtype),
        scratch_shapes=[
            pltpu.MemorySpace.VMEM((2, blk, dim), jnp.float32),
            pltpu.SemaphoreType.DMA((n,)),
            pltpu.SemaphoreType.DMA((n, batch)),
        ],
    )(emb_weight)
